# baseline (speedup 1.0000x reference)
import jax
import jax.numpy as jnp
from jax import lax
from jax.experimental import pallas as pl
from jax.experimental.pallas import tpu as pltpu
from jax.experimental.pallas import tpu_sc as plsc

VOCAB_SIZE = 1000000
SEQ_LEN = 200
EMBED_DIM = 32
BATCH = 4096

NC = 2
NS = 16
NW = NC * NS

G = 128
NSB = 8
NCHUNK = SEQ_LEN // NSB


def _sc_body(idxT_hbm, tok_hbm, posx_hbm, out_hbm,
             idx_v, bounce_v, out_v, posx_v, sem):
    wid = lax.axis_index("s") * NC + lax.axis_index("c")
    b0 = wid * G

    iotas = [lax.iota(jnp.int32, 16) + bb * 16 for bb in range(8)]

    @pl.loop(0, NCHUNK)
    def _chunk(c):
        s0 = c * NSB
        pltpu.sync_copy(idxT_hbm.at[pl.ds(s0, NSB), pl.ds(b0, G)], idx_v)
        pltpu.sync_copy(posx_hbm.at[pl.ds(s0, NSB)], posx_v)

        descs = [
            pltpu.async_copy(
                tok_hbm.at[idx_v.at[sb]], bounce_v.at[pl.ds(sb * G, G)], sem
            )
            for sb in range(NSB)
        ]
        for d in descs:
            d.wait()

        for sb in range(NSB):

            @pl.loop(0, EMBED_DIM)
            def _comp(d):
                pv = posx_v[sb, d, 0:16]
                dvec = jnp.full((16,), 0, jnp.int32) + d
                for bb in range(8):
                    rows = iotas[bb] + sb * G
                    v = plsc.load_gather(bounce_v, [rows, dvec])
                    out_v[sb, d, pl.ds(bb * 16, 16)] = v + pv

        pltpu.sync_copy(out_v, out_hbm.at[pl.ds(s0, NSB), :, pl.ds(b0, G)])


@jax.jit
def _sc_embed(idxT, token_table, posx):
    mesh = plsc.VectorSubcoreMesh(
        core_axis_name="c", subcore_axis_name="s", num_cores=NC, num_subcores=NS
    )
    return pl.kernel(
        _sc_body,
        out_type=jax.ShapeDtypeStruct((SEQ_LEN, EMBED_DIM, BATCH), jnp.float32),
        mesh=mesh,
        scratch_types=[
            pltpu.VMEM((NSB, G), jnp.int32),
            pltpu.VMEM((NSB * G, EMBED_DIM), jnp.float32),
            pltpu.VMEM((NSB, EMBED_DIM, G), jnp.float32),
            pltpu.VMEM((NSB, EMBED_DIM, 16), jnp.float32),
            pltpu.SemaphoreType.DMA,
        ],
        compiler_params=pltpu.CompilerParams(
            use_tc_tiling_on_sc=False, needs_layout_passes=False
        ),
    )(idxT, token_table, posx)


def kernel(inputs, token_table, position_table):
    idxT = inputs.astype(jnp.int32).T
    posx = jnp.broadcast_to(position_table[:, :, None], (SEQ_LEN, EMBED_DIM, 16))
    out = _sc_embed(idxT, token_table, posx)
    return jnp.transpose(out, (2, 0, 1))


# speedup vs baseline: 1.0486x; 1.0486x over previous
import jax
import jax.numpy as jnp
from jax import lax
from jax.experimental import pallas as pl
from jax.experimental.pallas import tpu as pltpu
from jax.experimental.pallas import tpu_sc as plsc

VOCAB_SIZE = 1000000
SEQ_LEN = 200
EMBED_DIM = 32
BATCH = 4096

NC = 2
NS = 16
NW = NC * NS

G = 128
NSB = 8
BPH = 4
NCHUNK = SEQ_LEN // NSB


def _sc_body(idxT_hbm, tok4_hbm, posx_hbm, out_hbm,
             idx_v, idx4_v, bounce_v, out_v, posx_v, sem):
    wid = lax.axis_index("s") * NC + lax.axis_index("c")
    b0 = wid * G
    iota = lax.iota(jnp.int32, 16)

    @pl.loop(0, NCHUNK)
    def _chunk(c):
        s0 = c * NSB
        pltpu.sync_copy(idxT_hbm.at[pl.ds(s0, NSB), pl.ds(b0, G)], idx_v)
        pltpu.sync_copy(posx_hbm.at[pl.ds(s0, NSB)], posx_v)

        for sb in range(NSB):
            for kk in range(G // 16):
                idx4_v[sb, pl.ds(kk * 16, 16)] = (
                    idx_v[sb, pl.ds(kk * 16, 16)] >> 2
                )

        for half in range(NSB // BPH):
            descs = [
                pltpu.async_copy(
                    tok4_hbm.at[idx4_v.at[half * BPH + k]],
                    bounce_v.at[pl.ds(k * G, G)],
                    sem,
                )
                for k in range(BPH)
            ]
            for dd in descs:
                dd.wait()

            for k in range(BPH):
                sb = half * BPH + k

                @pl.loop(0, EMBED_DIM)
                def _comp(d):
                    diag = iota + d
                    diag = jnp.where(diag >= EMBED_DIM, diag - EMBED_DIM, diag)
                    pv = posx_v[sb, pl.ds(d * 16, 16)]
                    sbv = jnp.full((16,), sb, jnp.int32)
                    for bb in range(G // 16):
                        raw = idx_v[sb, pl.ds(bb * 16, 16)]
                        col = ((raw & 3) << 5) + diag
                        rows = iota + (k * G + bb * 16)
                        v = plsc.load_gather(bounce_v, [rows, col])
                        plsc.store_scatter(
                            out_v, [sbv, diag, iota + bb * 16], v + pv
                        )

        pltpu.sync_copy(out_v, out_hbm.at[pl.ds(s0, NSB), :, pl.ds(b0, G)])


@jax.jit
def _sc_embed(idxT, tok4, posx):
    mesh = plsc.VectorSubcoreMesh(
        core_axis_name="c", subcore_axis_name="s", num_cores=NC, num_subcores=NS
    )
    return pl.kernel(
        _sc_body,
        out_type=jax.ShapeDtypeStruct((SEQ_LEN, EMBED_DIM, BATCH), jnp.float32),
        mesh=mesh,
        scratch_types=[
            pltpu.VMEM((NSB, G), jnp.int32),
            pltpu.VMEM((NSB, G), jnp.int32),
            pltpu.VMEM((BPH * G, 128), jnp.float32),
            pltpu.VMEM((NSB, EMBED_DIM, G), jnp.float32),
            pltpu.VMEM((NSB, EMBED_DIM * 16), jnp.float32),
            pltpu.SemaphoreType.DMA,
        ],
        compiler_params=pltpu.CompilerParams(
            use_tc_tiling_on_sc=True, needs_layout_passes=False
        ),
    )(idxT, tok4, posx)


def kernel(inputs, token_table, position_table):
    idxT = inputs.astype(jnp.int32).T
    tok4 = token_table.reshape(VOCAB_SIZE // 4, 4 * EMBED_DIM)
    comp = (jnp.arange(EMBED_DIM)[:, None] + jnp.arange(16)[None, :]) % EMBED_DIM
    posx = position_table[:, comp].reshape(SEQ_LEN, EMBED_DIM * 16)
    out = _sc_embed(idxT, tok4, posx)
    return jnp.transpose(out, (2, 0, 1))


# speedup vs baseline: 1.5292x; 1.4583x over previous
import jax
import jax.numpy as jnp
from jax import lax
from jax.experimental import pallas as pl
from jax.experimental.pallas import tpu as pltpu
from jax.experimental.pallas import tpu_sc as plsc

VOCAB_SIZE = 1000000
SEQ_LEN = 200
EMBED_DIM = 32
BATCH = 4096

NC = 2
NS = 16
NW = NC * NS

G = 128
NSB = 8
BPH = 4
NCHUNK = SEQ_LEN // NSB


def _sc_body(idxT_hbm, tok4_hbm, posx_hbm, out_hbm,
             idx_v, idx4_v, bounce_v, out_v, posx_v, sem):
    wid = lax.axis_index("s") * NC + lax.axis_index("c")
    b0 = wid * G
    iota = lax.iota(jnp.int32, 16)

    @pl.loop(0, NCHUNK)
    def _chunk(c):
        s0 = c * NSB
        pltpu.sync_copy(idxT_hbm.at[pl.ds(s0, NSB), pl.ds(b0, G)], idx_v)
        pltpu.sync_copy(posx_hbm.at[pl.ds(s0, NSB)], posx_v)

        for sb in range(NSB):
            for kk in range(G // 16):
                idx4_v[sb, pl.ds(kk * 16, 16)] = (
                    idx_v[sb, pl.ds(kk * 16, 16)] >> 2
                )

        for half in range(NSB // BPH):
            descs = [
                pltpu.async_copy(
                    tok4_hbm.at[idx4_v.at[half * BPH + k]],
                    bounce_v.at[pl.ds(k * G, G)],
                    sem,
                )
                for k in range(BPH)
            ]
            for dd in descs:
                dd.wait()

            for k in range(BPH):
                sb = half * BPH + k

                @plsc.parallel_loop(0, EMBED_DIM, unroll=2)
                def _comp(d):
                    diag = iota + d
                    diag = jnp.where(diag >= EMBED_DIM, diag - EMBED_DIM, diag)
                    pv = posx_v[sb, pl.ds(d * 16, 16)]
                    sbv = jnp.full((16,), sb, jnp.int32)
                    for bb in range(G // 16):
                        raw = idx_v[sb, pl.ds(bb * 16, 16)]
                        col = ((raw & 3) << 5) + diag
                        rows = iota + (k * G + bb * 16)
                        v = plsc.load_gather(bounce_v, [rows, col])
                        plsc.store_scatter(
                            out_v, [sbv, diag, iota + bb * 16], v + pv
                        )

        pltpu.sync_copy(out_v, out_hbm.at[pl.ds(s0, NSB), :, pl.ds(b0, G)])


@jax.jit
def _sc_embed(idxT, tok4, posx):
    mesh = plsc.VectorSubcoreMesh(
        core_axis_name="c", subcore_axis_name="s", num_cores=NC, num_subcores=NS
    )
    return pl.kernel(
        _sc_body,
        out_type=jax.ShapeDtypeStruct((SEQ_LEN, EMBED_DIM, BATCH), jnp.float32),
        mesh=mesh,
        scratch_types=[
            pltpu.VMEM((NSB, G), jnp.int32),
            pltpu.VMEM((NSB, G), jnp.int32),
            pltpu.VMEM((BPH * G, 128), jnp.float32),
            pltpu.VMEM((NSB, EMBED_DIM, G), jnp.float32),
            pltpu.VMEM((NSB, EMBED_DIM * 16), jnp.float32),
            pltpu.SemaphoreType.DMA,
        ],
        compiler_params=pltpu.CompilerParams(
            use_tc_tiling_on_sc=True, needs_layout_passes=False
        ),
    )(idxT, tok4, posx)


def kernel(inputs, token_table, position_table):
    idxT = inputs.astype(jnp.int32).T
    tok4 = token_table.reshape(VOCAB_SIZE // 4, 4 * EMBED_DIM)
    comp = (jnp.arange(EMBED_DIM)[:, None] + jnp.arange(16)[None, :]) % EMBED_DIM
    posx = position_table[:, comp].reshape(SEQ_LEN, EMBED_DIM * 16)
    out = _sc_embed(idxT, tok4, posx)
    return jnp.transpose(out, (2, 0, 1))
